# baseline (speedup 1.0000x reference)
import functools

import jax
import jax.numpy as jnp
from jax import lax
from jax.experimental import pallas as pl
from jax.experimental.pallas import tpu as pltpu
from jax.experimental.pallas import tpu_sc as plsc

NC = 2
NS = 16
NL = 16
NW = NC * NS


def _make_agg(N, D, E, want_deg):
    EW = E // NW
    K = 40
    NCHUNK = EW // K
    NP = N
    RPT = (N // NS) // 8 * 8
    RLAST = N - (NS - 1) * RPT

    mesh = plsc.VectorSubcoreMesh(core_axis_name="c", subcore_axis_name="s",
                                  num_cores=NC, num_subcores=NS)

    out_type = [jax.ShapeDtypeStruct((NC, N, D), jnp.float32)]
    if want_deg:
        out_type.append(jax.ShapeDtypeStruct((NC, N, 8), jnp.float32))

    NB = 5
    GA = NB - 2
    scratch = [
        pltpu.VMEM((NCHUNK, K), jnp.int32),
        pltpu.VMEM((NCHUNK, K), jnp.int32),
    ]
    scratch += [pltpu.VMEM((K, D), jnp.float32) for _ in range(NB)]
    scratch += [pltpu.SemaphoreType.DMA for _ in range(NB)]
    scratch += [pltpu.SemaphoreType.DMA for _ in range(NB)]
    scratch += [pltpu.VMEM_SHARED((N, D), jnp.float32)]
    if want_deg:
        scratch += [pltpu.SemaphoreType.DMA for _ in range(NB)]
    if want_deg:
        scratch.append(pltpu.VMEM((K, 8), jnp.float32))
        scratch.append(pltpu.VMEM_SHARED((N, 8), jnp.float32))

    @functools.partial(
        pl.kernel, mesh=mesh, out_type=out_type, scratch_types=scratch,
        compiler_params=pltpu.CompilerParams(use_tc_tiling_on_sc=False))
    def agg(*refs):
        if want_deg:
            (feat, src3, dst3, z2d, z1d, ones_h, part, deg_out,
             src_v, dst_v) = refs[:10]
            rows = refs[10:10 + NB]
            gsem = refs[10 + NB:10 + 2 * NB]
            ssem = refs[10 + 2 * NB:10 + 3 * NB]
            acc_s = refs[10 + 3 * NB]
            dsem = refs[11 + 3 * NB:11 + 4 * NB]
            ones_v, dacc_s = refs[11 + 4 * NB:]
        else:
            (feat, src3, dst3, z2d, part, src_v, dst_v) = refs[:7]
            rows = refs[7:7 + NB]
            gsem = refs[7 + NB:7 + 2 * NB]
            ssem = refs[7 + 2 * NB:7 + 3 * NB]
            acc_s = refs[7 + 3 * NB]
        c = lax.axis_index("c")
        s = lax.axis_index("s")
        wid = s * NC + c

        @pl.when(s < NS - 1)
        def _():
            pltpu.sync_copy(z2d.at[pl.ds(0, RPT)], acc_s.at[pl.ds(s * RPT, RPT)])
            if want_deg:
                pltpu.sync_copy(z1d.at[pl.ds(0, RPT)],
                                dacc_s.at[pl.ds(s * RPT, RPT)])

        @pl.when(s == NS - 1)
        def _():
            pltpu.sync_copy(z2d, acc_s.at[pl.ds((NS - 1) * RPT, RLAST)])
            if want_deg:
                pltpu.sync_copy(z1d, dacc_s.at[pl.ds((NS - 1) * RPT, RLAST)])

        if want_deg:
            pltpu.sync_copy(ones_h, ones_v)

        pltpu.sync_copy(src3.at[wid], src_v)
        pltpu.sync_copy(dst3.at[wid], dst_v)
        plsc.subcore_barrier()

        def fire_gather(j, b):
            pltpu.async_copy(feat.at[src_v.at[j]], rows[b], gsem[b])

        def wait_scatter(b):
            pltpu.make_async_copy(rows[b], acc_s.at[dst_v.at[0]],
                                  ssem[b]).wait()
            if want_deg:
                pltpu.make_async_copy(ones_v, dacc_s.at[dst_v.at[0]],
                                      dsem[b]).wait()

        def handle(j, b, traced):
            pltpu.make_async_copy(feat.at[src_v.at[j]], rows[b],
                                  gsem[b]).wait()
            pltpu.async_copy(rows[b], acc_s.at[dst_v.at[j]], ssem[b],
                             add=True)
            if want_deg:
                pltpu.async_copy(ones_v, dacc_s.at[dst_v.at[j]], dsem[b],
                                 add=True)
            b2 = (b + GA) % NB
            if traced:

                @pl.when(j >= NB - GA)
                def _():
                    wait_scatter(b2)

                fire_gather(j + GA, b2)
            elif j + GA < NCHUNK:
                wait_scatter(b2)
                fire_gather(j + GA, b2)

        for j0 in range(GA):
            fire_gather(j0, j0)
        NMAIN = (NCHUNK - GA) // NB

        def chunk(p, carry):
            for b in range(NB):
                handle(p * NB + b, b, True)
            return carry

        lax.fori_loop(0, NMAIN, chunk, 0)
        for j in range(NMAIN * NB, NCHUNK):
            handle(j, j % NB, False)
        for j in range(NCHUNK - NB, NCHUNK):
            b = j % NB
            pltpu.make_async_copy(rows[b], acc_s.at[dst_v.at[0]],
                                  ssem[b]).wait()
            if want_deg:
                pltpu.make_async_copy(ones_v, dacc_s.at[dst_v.at[0]],
                                      dsem[b]).wait()
        plsc.subcore_barrier()

        @pl.when(s < NS - 1)
        def _():
            pltpu.sync_copy(acc_s.at[pl.ds(s * RPT, RPT)],
                            part.at[c].at[pl.ds(s * RPT, RPT)])

        @pl.when(s == NS - 1)
        def _():
            pltpu.sync_copy(acc_s.at[pl.ds((NS - 1) * RPT, RLAST)],
                            part.at[c].at[pl.ds((NS - 1) * RPT, RLAST)])
        if want_deg:

            @pl.when(s == 0)
            def _():
                pltpu.sync_copy(dacc_s, deg_out.at[c])

    return agg


def _linear_body(xin_ref, wT_ref, out_ref):
    out_ref[...] = jnp.dot(xin_ref[...], wT_ref[...],
                           preferred_element_type=jnp.float32)


def _linear(xin, wT):
    N, D = xin.shape
    R = 1000
    return pl.pallas_call(
        _linear_body,
        grid=(N // R,),
        in_specs=[
            pl.BlockSpec((R, D), lambda i: (i, 0)),
            pl.BlockSpec((D, D), lambda i: (0, 0)),
        ],
        out_specs=pl.BlockSpec((R, D), lambda i: (i, 0)),
        out_shape=jax.ShapeDtypeStruct((N, D), jnp.float32),
    )(xin, wT)


def _dense_body(relu, part_ref, degp_ref, r_ref, wlT_ref,
                scale_ref, shift_ref, out_ref):
    deg = jnp.maximum(degp_ref[0] + degp_ref[1], 1.0)
    agg = (part_ref[0] + part_ref[1]) / deg
    z = (jnp.dot(agg, wlT_ref[...], preferred_element_type=jnp.float32)
         + r_ref[...])
    z = z * scale_ref[...] + shift_ref[...]
    if relu:
        z = jnp.maximum(z, 0.0)
    out_ref[...] = z


def _dense(part, degp, r, wlT, scale, shift, relu):
    N, D = r.shape
    R = 1000
    grid = N // R
    return pl.pallas_call(
        functools.partial(_dense_body, relu),
        grid=(grid,),
        in_specs=[
            pl.BlockSpec((NC, R, D), lambda i: (0, i, 0)),
            pl.BlockSpec((NC, R, 1), lambda i: (0, i, 0)),
            pl.BlockSpec((R, D), lambda i: (i, 0)),
            pl.BlockSpec((D, D), lambda i: (0, 0)),
            pl.BlockSpec((1, D), lambda i: (0, 0)),
            pl.BlockSpec((1, D), lambda i: (0, 0)),
        ],
        out_specs=pl.BlockSpec((R, D), lambda i: (i, 0)),
        out_shape=jax.ShapeDtypeStruct((N, D), jnp.float32),
    )(part, degp, r, wlT, scale, shift)


def kernel(x, edge_index, depth, W_l0, b_l0, W_r0, bn_gamma, bn_beta,
           W_l1, b_l1, W_r1):
    N, D = x.shape
    E = edge_index.shape[1]
    EW = E // NW
    K = 40
    NCHUNK = EW // K

    src3 = edge_index[0].reshape(NW, NCHUNK, K)
    dst3 = edge_index[1].reshape(NW, NCHUNK, K)
    RPT = (N // NS) // 8 * 8
    RLAST = N - (NW // NC - 1) * RPT
    z2d = jnp.zeros((RLAST, D), jnp.float32)
    z1d = jnp.zeros((RLAST, 8), jnp.float32)
    ones_h = jnp.ones((K, 8), jnp.float32)

    agg0 = _make_agg(N, D, E, True)
    agg1 = _make_agg(N, D, E, False)

    r0 = _linear(x, W_r0.T)
    part0, deg = agg0(x, src3, dst3, z2d, z1d, ones_h)
    degp = deg[:, :, 0].reshape(NC, N, 1)

    g = (bn_gamma / jnp.sqrt(1.0 + 1e-5)).reshape(1, D)
    scale0 = g
    shift0 = (b_l0.reshape(1, D)) * g + bn_beta.reshape(1, D)
    h = _dense(part0, degp, r0, W_l0.T, scale0, shift0, True)

    r1 = _linear(h, W_r1.T)
    (part1,) = agg1(h, src3, dst3, z2d)
    ones = jnp.ones((1, D), jnp.float32)
    out = _dense(part1, degp, r1, W_l1.T, ones,
                 b_l1.reshape(1, D), False)
    return out


# speedup vs baseline: 1.0920x; 1.0920x over previous
import functools

import jax
import jax.numpy as jnp
from jax import lax
from jax.experimental import pallas as pl
from jax.experimental.pallas import tpu as pltpu
from jax.experimental.pallas import tpu_sc as plsc

NC = 2
NS = 16
NL = 16
NW = NC * NS


def _make_agg(N, D, E, want_deg):
    EW = E // NW
    K = 40
    NCHUNK = EW // K
    NP = N
    RPT = (N // NS) // 8 * 8
    RLAST = N - (NS - 1) * RPT

    mesh = plsc.VectorSubcoreMesh(core_axis_name="c", subcore_axis_name="s",
                                  num_cores=NC, num_subcores=NS)

    out_type = [jax.ShapeDtypeStruct((NC, N, D), jnp.float32)]
    if want_deg:
        out_type.append(jax.ShapeDtypeStruct((NC, N, 8), jnp.float32))

    NB = 5
    GA = NB - 1
    scratch = [
        pltpu.VMEM((NCHUNK, K), jnp.int32),
        pltpu.VMEM((NCHUNK, K), jnp.int32),
    ]
    scratch += [pltpu.VMEM((K, D), jnp.float32) for _ in range(NB)]
    scratch += [pltpu.SemaphoreType.DMA for _ in range(NB)]
    scratch += [pltpu.SemaphoreType.DMA for _ in range(NB)]
    scratch += [pltpu.VMEM_SHARED((N, D), jnp.float32)]
    if want_deg:
        scratch += [pltpu.SemaphoreType.DMA for _ in range(NB)]
    if want_deg:
        scratch.append(pltpu.VMEM((K, 8), jnp.float32))
        scratch.append(pltpu.VMEM_SHARED((N, 8), jnp.float32))

    @functools.partial(
        pl.kernel, mesh=mesh, out_type=out_type, scratch_types=scratch,
        compiler_params=pltpu.CompilerParams(use_tc_tiling_on_sc=False))
    def agg(*refs):
        if want_deg:
            (feat, src3, dst3, z2d, z1d, ones_h, part, deg_out,
             src_v, dst_v) = refs[:10]
            rows = refs[10:10 + NB]
            gsem = refs[10 + NB:10 + 2 * NB]
            ssem = refs[10 + 2 * NB:10 + 3 * NB]
            acc_s = refs[10 + 3 * NB]
            dsem = refs[11 + 3 * NB:11 + 4 * NB]
            ones_v, dacc_s = refs[11 + 4 * NB:]
        else:
            (feat, src3, dst3, z2d, part, src_v, dst_v) = refs[:7]
            rows = refs[7:7 + NB]
            gsem = refs[7 + NB:7 + 2 * NB]
            ssem = refs[7 + 2 * NB:7 + 3 * NB]
            acc_s = refs[7 + 3 * NB]
        c = lax.axis_index("c")
        s = lax.axis_index("s")
        wid = s * NC + c

        @pl.when(s < NS - 1)
        def _():
            pltpu.sync_copy(z2d.at[pl.ds(0, RPT)], acc_s.at[pl.ds(s * RPT, RPT)])
            if want_deg:
                pltpu.sync_copy(z1d.at[pl.ds(0, RPT)],
                                dacc_s.at[pl.ds(s * RPT, RPT)])

        @pl.when(s == NS - 1)
        def _():
            pltpu.sync_copy(z2d, acc_s.at[pl.ds((NS - 1) * RPT, RLAST)])
            if want_deg:
                pltpu.sync_copy(z1d, dacc_s.at[pl.ds((NS - 1) * RPT, RLAST)])

        if want_deg:
            pltpu.sync_copy(ones_h, ones_v)

        pltpu.sync_copy(src3.at[wid], src_v)
        pltpu.sync_copy(dst3.at[wid], dst_v)
        plsc.subcore_barrier()

        def fire_gather(j, b):
            pltpu.async_copy(feat.at[src_v.at[j]], rows[b], gsem[b])

        def wait_scatter(b):
            pltpu.make_async_copy(rows[b], acc_s.at[dst_v.at[0]],
                                  ssem[b]).wait()
            if want_deg:
                pltpu.make_async_copy(ones_v, dacc_s.at[dst_v.at[0]],
                                      dsem[b]).wait()

        def handle(j, b, traced):
            pltpu.make_async_copy(feat.at[src_v.at[j]], rows[b],
                                  gsem[b]).wait()
            pltpu.async_copy(rows[b], acc_s.at[dst_v.at[j]], ssem[b],
                             add=True)
            if want_deg:
                pltpu.async_copy(ones_v, dacc_s.at[dst_v.at[j]], dsem[b],
                                 add=True)
            b2 = (b + GA) % NB
            if traced:

                @pl.when(j >= NB - GA)
                def _():
                    wait_scatter(b2)

                fire_gather(j + GA, b2)
            elif j + GA < NCHUNK:
                wait_scatter(b2)
                fire_gather(j + GA, b2)

        for j0 in range(GA):
            fire_gather(j0, j0)
        NMAIN = (NCHUNK - GA) // NB

        def chunk(p, carry):
            for b in range(NB):
                handle(p * NB + b, b, True)
            return carry

        lax.fori_loop(0, NMAIN, chunk, 0)
        for j in range(NMAIN * NB, NCHUNK):
            handle(j, j % NB, False)
        for j in range(NCHUNK - NB, NCHUNK):
            b = j % NB
            pltpu.make_async_copy(rows[b], acc_s.at[dst_v.at[0]],
                                  ssem[b]).wait()
            if want_deg:
                pltpu.make_async_copy(ones_v, dacc_s.at[dst_v.at[0]],
                                      dsem[b]).wait()
        plsc.subcore_barrier()

        @pl.when(s < NS - 1)
        def _():
            pltpu.sync_copy(acc_s.at[pl.ds(s * RPT, RPT)],
                            part.at[c].at[pl.ds(s * RPT, RPT)])

        @pl.when(s == NS - 1)
        def _():
            pltpu.sync_copy(acc_s.at[pl.ds((NS - 1) * RPT, RLAST)],
                            part.at[c].at[pl.ds((NS - 1) * RPT, RLAST)])
        if want_deg:

            @pl.when(s == 0)
            def _():
                pltpu.sync_copy(dacc_s, deg_out.at[c])

    return agg


def _dense_body(relu, part_ref, degp_ref, xin_ref, wlT_ref, wrT_ref,
                scale_ref, shift_ref, out_ref):
    deg = jnp.maximum(degp_ref[0] + degp_ref[1], 1.0)
    agg = (part_ref[0] + part_ref[1]) / deg
    z = (jnp.dot(agg, wlT_ref[...], preferred_element_type=jnp.float32)
         + jnp.dot(xin_ref[...], wrT_ref[...],
                   preferred_element_type=jnp.float32))
    z = z * scale_ref[...] + shift_ref[...]
    if relu:
        z = jnp.maximum(z, 0.0)
    out_ref[...] = z


def _dense(part, degp, xin, wlT, wrT, scale, shift, relu):
    N, D = xin.shape
    R = 1000
    grid = N // R
    return pl.pallas_call(
        functools.partial(_dense_body, relu),
        grid=(grid,),
        in_specs=[
            pl.BlockSpec((NC, R, D), lambda i: (0, i, 0)),
            pl.BlockSpec((NC, R, 1), lambda i: (0, i, 0)),
            pl.BlockSpec((R, D), lambda i: (i, 0)),
            pl.BlockSpec((D, D), lambda i: (0, 0)),
            pl.BlockSpec((D, D), lambda i: (0, 0)),
            pl.BlockSpec((1, D), lambda i: (0, 0)),
            pl.BlockSpec((1, D), lambda i: (0, 0)),
        ],
        out_specs=pl.BlockSpec((R, D), lambda i: (i, 0)),
        out_shape=jax.ShapeDtypeStruct((N, D), jnp.float32),
    )(part, degp, xin, wlT, wrT, scale, shift)


def kernel(x, edge_index, depth, W_l0, b_l0, W_r0, bn_gamma, bn_beta,
           W_l1, b_l1, W_r1):
    N, D = x.shape
    E = edge_index.shape[1]
    EW = E // NW
    K = 40
    NCHUNK = EW // K

    src3 = edge_index[0].reshape(NW, NCHUNK, K)
    dst3 = edge_index[1].reshape(NW, NCHUNK, K)
    RPT = (N // NS) // 8 * 8
    RLAST = N - (NW // NC - 1) * RPT
    z2d = jnp.zeros((RLAST, D), jnp.float32)
    z1d = jnp.zeros((RLAST, 8), jnp.float32)
    ones_h = jnp.ones((K, 8), jnp.float32)

    agg0 = _make_agg(N, D, E, True)
    agg1 = _make_agg(N, D, E, False)

    part0, deg = agg0(x, src3, dst3, z2d, z1d, ones_h)
    degp = deg[:, :, 0].reshape(NC, N, 1)

    g = (bn_gamma / jnp.sqrt(1.0 + 1e-5)).reshape(1, D)
    scale0 = g
    shift0 = (b_l0.reshape(1, D)) * g + bn_beta.reshape(1, D)
    h = _dense(part0, degp, x, W_l0.T, W_r0.T, scale0, shift0, True)

    (part1,) = agg1(h, src3, dst3, z2d)
    ones = jnp.ones((1, D), jnp.float32)
    out = _dense(part1, degp, h, W_l1.T, W_r1.T, ones,
                 b_l1.reshape(1, D), False)
    return out


# speedup vs baseline: 1.1090x; 1.0156x over previous
import functools

import jax
import jax.numpy as jnp
from jax import lax
from jax.experimental import pallas as pl
from jax.experimental.pallas import tpu as pltpu
from jax.experimental.pallas import tpu_sc as plsc

NC = 2
NS = 16
NL = 16
NW = NC * NS


def _make_agg(N, D, E, want_deg):
    EW = E // NW
    K = 40
    NCHUNK = EW // K
    NP = N
    RPT = (N // NS) // 8 * 8
    RLAST = N - (NS - 1) * RPT

    mesh = plsc.VectorSubcoreMesh(core_axis_name="c", subcore_axis_name="s",
                                  num_cores=NC, num_subcores=NS)

    out_type = [jax.ShapeDtypeStruct((NC, N, D), jnp.float32)]
    if want_deg:
        out_type.append(jax.ShapeDtypeStruct((NC, N, 8), jnp.float32))

    NB = 5
    GA = NB - 1
    scratch = [
        pltpu.VMEM((NCHUNK, K), jnp.int32),
        pltpu.VMEM((NCHUNK, K), jnp.int32),
    ]
    scratch += [pltpu.VMEM((K, D), jnp.float32) for _ in range(NB)]
    scratch += [pltpu.SemaphoreType.DMA for _ in range(NB)]
    scratch += [pltpu.SemaphoreType.DMA for _ in range(NB)]
    scratch += [pltpu.VMEM_SHARED((N, D), jnp.float32)]
    if want_deg:
        scratch += [pltpu.SemaphoreType.DMA for _ in range(NB)]
    if want_deg:
        scratch.append(pltpu.VMEM((K, 8), jnp.float32))
        scratch.append(pltpu.VMEM_SHARED((N, 8), jnp.float32))

    @functools.partial(
        pl.kernel, mesh=mesh, out_type=out_type, scratch_types=scratch,
        compiler_params=pltpu.CompilerParams(use_tc_tiling_on_sc=False))
    def agg(*refs):
        if want_deg:
            (feat, src3, dst3, z2d, z1d, ones_h, part, deg_out,
             src_v, dst_v) = refs[:10]
            rows = refs[10:10 + NB]
            gsem = refs[10 + NB:10 + 2 * NB]
            ssem = refs[10 + 2 * NB:10 + 3 * NB]
            acc_s = refs[10 + 3 * NB]
            dsem = refs[11 + 3 * NB:11 + 4 * NB]
            ones_v, dacc_s = refs[11 + 4 * NB:]
        else:
            (feat, src3, dst3, z2d, part, src_v, dst_v) = refs[:7]
            rows = refs[7:7 + NB]
            gsem = refs[7 + NB:7 + 2 * NB]
            ssem = refs[7 + 2 * NB:7 + 3 * NB]
            acc_s = refs[7 + 3 * NB]
        c = lax.axis_index("c")
        s = lax.axis_index("s")
        wid = s * NC + c

        @pl.when(s < NS - 1)
        def _():
            pltpu.async_copy(z2d.at[pl.ds(0, RPT)],
                             acc_s.at[pl.ds(s * RPT, RPT)], ssem[0])
            if want_deg:
                pltpu.async_copy(z1d.at[pl.ds(0, RPT)],
                                 dacc_s.at[pl.ds(s * RPT, RPT)], ssem[1])

        @pl.when(s == NS - 1)
        def _():
            pltpu.async_copy(z2d, acc_s.at[pl.ds((NS - 1) * RPT, RLAST)],
                             ssem[0])
            if want_deg:
                pltpu.async_copy(z1d, dacc_s.at[pl.ds((NS - 1) * RPT, RLAST)],
                                 ssem[1])

        if want_deg:
            pltpu.async_copy(ones_h, ones_v, ssem[2])

        pltpu.async_copy(src3.at[wid], src_v, ssem[3])
        pltpu.async_copy(dst3.at[wid], dst_v, ssem[4])

        @pl.when(s < NS - 1)
        def _():
            pltpu.make_async_copy(z2d.at[pl.ds(0, RPT)],
                                  acc_s.at[pl.ds(s * RPT, RPT)],
                                  ssem[0]).wait()
            if want_deg:
                pltpu.make_async_copy(z1d.at[pl.ds(0, RPT)],
                                      dacc_s.at[pl.ds(s * RPT, RPT)],
                                      ssem[1]).wait()

        @pl.when(s == NS - 1)
        def _():
            pltpu.make_async_copy(z2d, acc_s.at[pl.ds((NS - 1) * RPT, RLAST)],
                                  ssem[0]).wait()
            if want_deg:
                pltpu.make_async_copy(z1d,
                                      dacc_s.at[pl.ds((NS - 1) * RPT, RLAST)],
                                      ssem[1]).wait()

        if want_deg:
            pltpu.make_async_copy(ones_h, ones_v, ssem[2]).wait()
        pltpu.make_async_copy(src3.at[wid], src_v, ssem[3]).wait()
        pltpu.make_async_copy(dst3.at[wid], dst_v, ssem[4]).wait()
        plsc.subcore_barrier()

        def fire_gather(j, b):
            pltpu.async_copy(feat.at[src_v.at[j]], rows[b], gsem[b])

        def wait_scatter(b):
            pltpu.make_async_copy(rows[b], acc_s.at[dst_v.at[0]],
                                  ssem[b]).wait()
            if want_deg:
                pltpu.make_async_copy(ones_v, dacc_s.at[dst_v.at[0]],
                                      dsem[b]).wait()

        def handle(j, b, traced):
            pltpu.make_async_copy(feat.at[src_v.at[j]], rows[b],
                                  gsem[b]).wait()
            pltpu.async_copy(rows[b], acc_s.at[dst_v.at[j]], ssem[b],
                             add=True)
            if want_deg:
                pltpu.async_copy(ones_v, dacc_s.at[dst_v.at[j]], dsem[b],
                                 add=True)
            b2 = (b + GA) % NB
            if traced:

                @pl.when(j >= NB - GA)
                def _():
                    wait_scatter(b2)

                fire_gather(j + GA, b2)
            elif j + GA < NCHUNK:
                wait_scatter(b2)
                fire_gather(j + GA, b2)

        for j0 in range(GA):
            fire_gather(j0, j0)
        NMAIN = (NCHUNK - GA) // NB

        def chunk(p, carry):
            for b in range(NB):
                handle(p * NB + b, b, True)
            return carry

        lax.fori_loop(0, NMAIN, chunk, 0)
        for j in range(NMAIN * NB, NCHUNK):
            handle(j, j % NB, False)
        for j in range(NCHUNK - NB, NCHUNK):
            b = j % NB
            pltpu.make_async_copy(rows[b], acc_s.at[dst_v.at[0]],
                                  ssem[b]).wait()
            if want_deg:
                pltpu.make_async_copy(ones_v, dacc_s.at[dst_v.at[0]],
                                      dsem[b]).wait()
        plsc.subcore_barrier()

        @pl.when(s < NS - 1)
        def _():
            pltpu.sync_copy(acc_s.at[pl.ds(s * RPT, RPT)],
                            part.at[c].at[pl.ds(s * RPT, RPT)])

        @pl.when(s == NS - 1)
        def _():
            pltpu.sync_copy(acc_s.at[pl.ds((NS - 1) * RPT, RLAST)],
                            part.at[c].at[pl.ds((NS - 1) * RPT, RLAST)])
        if want_deg:

            @pl.when(s == 0)
            def _():
                pltpu.sync_copy(dacc_s, deg_out.at[c])

    return agg


def _dense_body(relu, part_ref, degp_ref, xin_ref, wlT_ref, wrT_ref,
                scale_ref, shift_ref, out_ref):
    deg = jnp.maximum(degp_ref[0] + degp_ref[1], 1.0)
    agg = (part_ref[0] + part_ref[1]) / deg
    z = (jnp.dot(agg, wlT_ref[...], preferred_element_type=jnp.float32)
         + jnp.dot(xin_ref[...], wrT_ref[...],
                   preferred_element_type=jnp.float32))
    z = z * scale_ref[...] + shift_ref[...]
    if relu:
        z = jnp.maximum(z, 0.0)
    out_ref[...] = z


def _dense(part, degp, xin, wlT, wrT, scale, shift, relu):
    N, D = xin.shape
    R = 1000
    grid = N // R
    return pl.pallas_call(
        functools.partial(_dense_body, relu),
        grid=(grid,),
        in_specs=[
            pl.BlockSpec((NC, R, D), lambda i: (0, i, 0)),
            pl.BlockSpec((NC, R, 1), lambda i: (0, i, 0)),
            pl.BlockSpec((R, D), lambda i: (i, 0)),
            pl.BlockSpec((D, D), lambda i: (0, 0)),
            pl.BlockSpec((D, D), lambda i: (0, 0)),
            pl.BlockSpec((1, D), lambda i: (0, 0)),
            pl.BlockSpec((1, D), lambda i: (0, 0)),
        ],
        out_specs=pl.BlockSpec((R, D), lambda i: (i, 0)),
        out_shape=jax.ShapeDtypeStruct((N, D), jnp.float32),
    )(part, degp, xin, wlT, wrT, scale, shift)


def kernel(x, edge_index, depth, W_l0, b_l0, W_r0, bn_gamma, bn_beta,
           W_l1, b_l1, W_r1):
    N, D = x.shape
    E = edge_index.shape[1]
    EW = E // NW
    K = 40
    NCHUNK = EW // K

    src3 = edge_index[0].reshape(NW, NCHUNK, K)
    dst3 = edge_index[1].reshape(NW, NCHUNK, K)
    RPT = (N // NS) // 8 * 8
    RLAST = N - (NW // NC - 1) * RPT
    z2d = jnp.zeros((RLAST, D), jnp.float32)
    z1d = jnp.zeros((RLAST, 8), jnp.float32)
    ones_h = jnp.ones((K, 8), jnp.float32)

    agg0 = _make_agg(N, D, E, True)
    agg1 = _make_agg(N, D, E, False)

    part0, deg = agg0(x, src3, dst3, z2d, z1d, ones_h)
    degp = deg[:, :, 0].reshape(NC, N, 1)

    g = (bn_gamma / jnp.sqrt(1.0 + 1e-5)).reshape(1, D)
    scale0 = g
    shift0 = (b_l0.reshape(1, D)) * g + bn_beta.reshape(1, D)
    h = _dense(part0, degp, x, W_l0.T, W_r0.T, scale0, shift0, True)

    (part1,) = agg1(h, src3, dst3, z2d)
    ones = jnp.ones((1, D), jnp.float32)
    out = _dense(part1, degp, h, W_l1.T, W_r1.T, ones,
                 b_l1.reshape(1, D), False)
    return out
